# baseline (speedup 1.0000x reference)
import functools

import jax
import jax.numpy as jnp
from jax import lax
from jax.experimental import pallas as pl
from jax.experimental.pallas import tpu as pltpu
from jax.experimental.pallas import tpu_sc as plsc

MAX_SEQ_LEN = 20480
D = 1024
MAX_LEN = 2048
B = 4

_INFO = plsc.get_sparse_core_info()
NC = _INFO.num_cores
NS = _INFO.num_subcores
L = _INFO.num_lanes
NW = NC * NS

ROWS = B * MAX_LEN
RPT = ROWS // NW
TPB = MAX_LEN // RPT
CH = 32
NCH = RPT // CH
PAD = 8


def _pe_body(len_hbm, table_hbm, zeros_hbm, out_hbm, len_v, idx_v,
             buf0, buf1, zbuf, gsem0, gsem1, psem0, psem1, zsem):
    wid = lax.axis_index("s") * NC + lax.axis_index("c")
    b = wid // TPB
    pos0 = (wid % TPB) * RPT
    row0 = wid * RPT

    pltpu.sync_copy(len_hbm.at[b], len_v)
    pltpu.sync_copy(zeros_hbm, zbuf)
    lenb = len_v[...]

    full = []
    empty = []
    bnd = []
    for c in range(NCH):
        s = pos0 + c * CH
        f = jnp.all(lenb >= s + CH)
        e = jnp.all(lenb <= s)
        full.append(f)
        empty.append(e)
        bnd.append(jnp.logical_not(f) & jnp.logical_not(e))

    lane = lax.iota(jnp.int32, L)
    for c in range(NCH):
        for i in range(CH // L):
            vals = lane + (pos0 + c * CH + i * L + 1)
            idx_v[c, pl.ds(i * L, L)] = jnp.where(vals <= lenb, vals, 0)

    bufs = (buf0, buf1)
    gsems = (gsem0, gsem1)
    psems = (psem0, psem1)

    def lin_fill(c):
        return pltpu.make_async_copy(
            table_hbm.at[pl.ds(pos0 + c * CH, CH + PAD)], bufs[c % 2],
            gsems[c % 2])

    def ind_fill(c):
        return pltpu.make_async_copy(
            table_hbm.at[idx_v.at[c]], bufs[c % 2].at[pl.ds(0, CH)],
            gsems[c % 2])

    def put_full(c):
        return pltpu.make_async_copy(
            bufs[c % 2].at[pl.ds(1, CH)],
            out_hbm.at[pl.ds(row0 + c * CH, CH)], psems[c % 2])

    def put_bnd(c):
        return pltpu.make_async_copy(
            bufs[c % 2].at[pl.ds(0, CH)],
            out_hbm.at[pl.ds(row0 + c * CH, CH)], psems[c % 2])

    def put_zero(c):
        return pltpu.make_async_copy(
            zbuf, out_hbm.at[pl.ds(row0 + c * CH, CH)], zsem)

    def fill(c):
        @pl.when(full[c])
        def _():
            lin_fill(c).start()

        @pl.when(bnd[c])
        def _():
            ind_fill(c).start()

    def wait_fill(c):
        @pl.when(full[c])
        def _():
            lin_fill(c).wait()

        @pl.when(bnd[c])
        def _():
            ind_fill(c).wait()

    def put(c):
        @pl.when(full[c])
        def _():
            put_full(c).start()

        @pl.when(bnd[c])
        def _():
            put_bnd(c).start()

        @pl.when(empty[c])
        def _():
            put_zero(c).start()

    def wait_put(c):
        @pl.when(jnp.logical_not(empty[c]))
        def _():
            put_bnd(c).wait()

    fill(0)
    for c in range(NCH):
        if c + 1 < NCH:
            if c >= 1:
                wait_put(c - 1)
            fill(c + 1)
        wait_fill(c)
        put(c)
    wait_put(NCH - 2)
    wait_put(NCH - 1)
    for c in range(NCH):
        @pl.when(empty[c])
        def _():
            put_zero(c).wait()


def kernel(input_len, pos_enc):
    len_bcast = jnp.broadcast_to(input_len.astype(jnp.int32)[:, None], (B, L))
    zeros = jnp.zeros((CH, D), jnp.float32)
    mesh = plsc.VectorSubcoreMesh(core_axis_name="c", subcore_axis_name="s")
    run = functools.partial(
        pl.kernel,
        mesh=mesh,
        out_type=jax.ShapeDtypeStruct((ROWS, D), jnp.float32),
        compiler_params=pltpu.CompilerParams(
            use_tc_tiling_on_sc=False, needs_layout_passes=False),
        scratch_types=[
            pltpu.VMEM((L,), jnp.int32),
            pltpu.VMEM((NCH, CH), jnp.int32),
            pltpu.VMEM((CH + PAD, D), jnp.float32),
            pltpu.VMEM((CH + PAD, D), jnp.float32),
            pltpu.VMEM((CH, D), jnp.float32),
            pltpu.SemaphoreType.DMA,
            pltpu.SemaphoreType.DMA,
            pltpu.SemaphoreType.DMA,
            pltpu.SemaphoreType.DMA,
            pltpu.SemaphoreType.DMA,
        ],
    )(_pe_body)
    out = run(len_bcast, pos_enc, zeros)
    return out.reshape(B, MAX_LEN, D)


# speedup vs baseline: 2.9938x; 2.9938x over previous
import functools

import jax
import jax.numpy as jnp
from jax import lax
from jax.experimental import pallas as pl
from jax.experimental.pallas import tpu as pltpu
from jax.experimental.pallas import tpu_sc as plsc

MAX_SEQ_LEN = 20480
D = 1024
MAX_LEN = 2048
B = 4

_INFO = plsc.get_sparse_core_info()
NC = _INFO.num_cores
NS = _INFO.num_subcores
L = _INFO.num_lanes
NW = NC * NS

ROWS = B * MAX_LEN
RPT = ROWS // NW
TPB = MAX_LEN // RPT
CH = 32
NCH = RPT // CH


def _pe_body(len_hbm, tabs_hbm, table_hbm, zeros_hbm, out_hbm, len_v, idx_v,
             buf0, buf1, zbuf, gsem0, gsem1, psem0, psem1, zsem):
    wid = lax.axis_index("s") * NC + lax.axis_index("c")
    b = wid // TPB
    pos0 = (wid % TPB) * RPT

    pltpu.sync_copy(len_hbm.at[b], len_v)
    pltpu.sync_copy(zeros_hbm, zbuf)
    lenb = len_v[...]

    full = []
    empty = []
    bnd = []
    for c in range(NCH):
        s = pos0 + c * CH
        f = jnp.all(lenb >= s + CH)
        e = jnp.all(lenb <= s)
        full.append(f)
        empty.append(e)
        bnd.append(jnp.logical_not(f) & jnp.logical_not(e))

    lane = lax.iota(jnp.int32, L)
    for c in range(NCH):
        for i in range(CH // L):
            vals = lane + (pos0 + c * CH + i * L + 1)
            idx_v[c, pl.ds(i * L, L)] = jnp.where(vals <= lenb, vals, 0)

    bufs = (buf0, buf1)
    gsems = (gsem0, gsem1)
    psems = (psem0, psem1)

    def lin_fill(c):
        return pltpu.make_async_copy(
            tabs_hbm.at[pl.ds(pos0 + c * CH, CH)], bufs[c % 2], gsems[c % 2])

    def ind_fill(c):
        return pltpu.make_async_copy(
            table_hbm.at[idx_v.at[c]], bufs[c % 2], gsems[c % 2])

    def put_data(c):
        return pltpu.make_async_copy(
            bufs[c % 2], out_hbm.at[b, pl.ds(pos0 + c * CH, CH)],
            psems[c % 2])

    def put_zero(c):
        return pltpu.make_async_copy(
            zbuf, out_hbm.at[b, pl.ds(pos0 + c * CH, CH)], zsem)

    def fill(c):
        @pl.when(full[c])
        def _():
            lin_fill(c).start()

        @pl.when(bnd[c])
        def _():
            ind_fill(c).start()

    def wait_fill(c):
        @pl.when(jnp.logical_not(empty[c]))
        def _():
            lin_fill(c).wait()

    def put(c):
        @pl.when(jnp.logical_not(empty[c]))
        def _():
            put_data(c).start()

        @pl.when(empty[c])
        def _():
            put_zero(c).start()

    def wait_put(c):
        @pl.when(jnp.logical_not(empty[c]))
        def _():
            put_data(c).wait()

    fill(0)
    for c in range(NCH):
        if c + 1 < NCH:
            if c >= 1:
                wait_put(c - 1)
            fill(c + 1)
        wait_fill(c)
        put(c)
    wait_put(NCH - 2)
    wait_put(NCH - 1)
    for c in range(NCH):
        @pl.when(empty[c])
        def _():
            put_zero(c).wait()


def kernel(input_len, pos_enc):
    len_bcast = jnp.broadcast_to(input_len.astype(jnp.int32)[:, None], (B, L))
    tab_shift = lax.slice(pos_enc, (1, 0), (MAX_LEN + 1, D))
    zeros = jnp.zeros((CH, D), jnp.float32)
    mesh = plsc.VectorSubcoreMesh(core_axis_name="c", subcore_axis_name="s")
    run = functools.partial(
        pl.kernel,
        mesh=mesh,
        out_type=jax.ShapeDtypeStruct((B, MAX_LEN, D), jnp.float32),
        compiler_params=pltpu.CompilerParams(needs_layout_passes=False),
        scratch_types=[
            pltpu.VMEM((L,), jnp.int32),
            pltpu.VMEM((NCH, CH), jnp.int32),
            pltpu.VMEM((CH, D), jnp.float32),
            pltpu.VMEM((CH, D), jnp.float32),
            pltpu.VMEM((CH, D), jnp.float32),
            pltpu.SemaphoreType.DMA,
            pltpu.SemaphoreType.DMA,
            pltpu.SemaphoreType.DMA,
            pltpu.SemaphoreType.DMA,
            pltpu.SemaphoreType.DMA,
        ],
    )(_pe_body)
    return run(len_bcast, tab_shift, pos_enc, zeros)


# speedup vs baseline: 3.0268x; 1.0110x over previous
import functools

import jax
import jax.numpy as jnp
from jax import lax
from jax.experimental import pallas as pl
from jax.experimental.pallas import tpu as pltpu
from jax.experimental.pallas import tpu_sc as plsc

MAX_SEQ_LEN = 20480
D = 1024
MAX_LEN = 2048
B = 4

_INFO = plsc.get_sparse_core_info()
NC = _INFO.num_cores
NS = _INFO.num_subcores
L = _INFO.num_lanes
NW = NC * NS

ROWS = B * MAX_LEN
RPT = ROWS // NW
TPB = MAX_LEN // RPT
CH = 32
NCH = RPT // CH
NBUF = 3
ZR = 16


def _pe_body(len_hbm, tabs_hbm, table_hbm, zeros_hbm, out_hbm, len_v, idx_v,
             buf0, buf1, buf2, zbuf, gsem0, gsem1, gsem2, psem0, psem1, psem2,
             zsem):
    wid = lax.axis_index("s") * NC + lax.axis_index("c")
    b = wid // TPB
    pos0 = (wid % TPB) * RPT

    pltpu.sync_copy(len_hbm.at[b], len_v)
    pltpu.sync_copy(zeros_hbm, zbuf)
    lenb = len_v[...]

    full = []
    empty = []
    bnd = []
    for c in range(NCH):
        s = pos0 + c * CH
        f = jnp.all(lenb >= s + CH)
        e = jnp.all(lenb <= s)
        full.append(f)
        empty.append(e)
        bnd.append(jnp.logical_not(f) & jnp.logical_not(e))

    lane = lax.iota(jnp.int32, L)
    for c in range(NCH):
        for i in range(CH // L):
            vals = lane + (pos0 + c * CH + i * L + 1)
            idx_v[c, pl.ds(i * L, L)] = jnp.where(vals <= lenb, vals, 0)

    bufs = (buf0, buf1, buf2)
    gsems = (gsem0, gsem1, gsem2)
    psems = (psem0, psem1, psem2)

    def lin_fill(c):
        return pltpu.make_async_copy(
            tabs_hbm.at[pl.ds(pos0 + c * CH, CH)], bufs[c % NBUF],
            gsems[c % NBUF])

    def ind_fill(c):
        return pltpu.make_async_copy(
            table_hbm.at[idx_v.at[c]], bufs[c % NBUF], gsems[c % NBUF])

    def put_data(c):
        return pltpu.make_async_copy(
            bufs[c % NBUF], out_hbm.at[b, pl.ds(pos0 + c * CH, CH)],
            psems[c % NBUF])

    def put_zero(c, h):
        return pltpu.make_async_copy(
            zbuf, out_hbm.at[b, pl.ds(pos0 + c * CH + h * ZR, ZR)], zsem)

    def fill(c):
        @pl.when(full[c])
        def _():
            lin_fill(c).start()

        @pl.when(bnd[c])
        def _():
            ind_fill(c).start()

    def wait_fill(c):
        @pl.when(jnp.logical_not(empty[c]))
        def _():
            lin_fill(c).wait()

    def put(c):
        @pl.when(jnp.logical_not(empty[c]))
        def _():
            put_data(c).start()

        @pl.when(empty[c])
        def _():
            put_zero(c, 0).start()
            put_zero(c, 1).start()

    def wait_put(c):
        @pl.when(jnp.logical_not(empty[c]))
        def _():
            put_data(c).wait()

    fill(0)
    fill(1)
    for c in range(NCH):
        if c + 2 < NCH:
            if c >= 1:
                wait_put(c - 1)
            fill(c + 2)
        wait_fill(c)
        put(c)
    wait_put(NCH - 3)
    wait_put(NCH - 2)
    wait_put(NCH - 1)
    for c in range(NCH):
        @pl.when(empty[c])
        def _():
            put_zero(c, 0).wait()
            put_zero(c, 1).wait()


def kernel(input_len, pos_enc):
    len_bcast = jnp.broadcast_to(input_len.astype(jnp.int32)[:, None], (B, L))
    tab_shift = lax.slice(pos_enc, (1, 0), (MAX_LEN + 1, D))
    zeros = jnp.zeros((ZR, D), jnp.float32)
    mesh = plsc.VectorSubcoreMesh(core_axis_name="c", subcore_axis_name="s")
    run = functools.partial(
        pl.kernel,
        mesh=mesh,
        out_type=jax.ShapeDtypeStruct((B, MAX_LEN, D), jnp.float32),
        compiler_params=pltpu.CompilerParams(needs_layout_passes=False),
        scratch_types=[
            pltpu.VMEM((L,), jnp.int32),
            pltpu.VMEM((NCH, CH), jnp.int32),
            pltpu.VMEM((CH, D), jnp.float32),
            pltpu.VMEM((CH, D), jnp.float32),
            pltpu.VMEM((CH, D), jnp.float32),
            pltpu.VMEM((ZR, D), jnp.float32),
            pltpu.SemaphoreType.DMA,
            pltpu.SemaphoreType.DMA,
            pltpu.SemaphoreType.DMA,
            pltpu.SemaphoreType.DMA,
            pltpu.SemaphoreType.DMA,
            pltpu.SemaphoreType.DMA,
            pltpu.SemaphoreType.DMA,
        ],
    )(_pe_body)
    return run(len_bcast, tab_shift, pos_enc, zeros)


# speedup vs baseline: 3.2656x; 1.0789x over previous
import functools

import jax
import jax.numpy as jnp
from jax import lax
from jax.experimental import pallas as pl
from jax.experimental.pallas import tpu as pltpu
from jax.experimental.pallas import tpu_sc as plsc

MAX_SEQ_LEN = 20480
D = 1024
MAX_LEN = 2048
B = 4

_INFO = plsc.get_sparse_core_info()
NC = _INFO.num_cores
NS = _INFO.num_subcores
L = _INFO.num_lanes
NW = NC * NS

ROWS = B * MAX_LEN
RPT = ROWS // NW
TPB = MAX_LEN // RPT
CH = 32
NCH = RPT // CH
NBUF = 3
ZR = 16


def _pe_body(len_hbm, tabs_hbm, table_hbm, out_hbm, len_v, idx_v,
             buf0, buf1, buf2, zbuf, gsem0, gsem1, gsem2, psem0, psem1, psem2,
             zsem):
    wid = lax.axis_index("s") * NC + lax.axis_index("c")
    b = wid // TPB
    pos0 = (wid % TPB) * RPT

    pltpu.sync_copy(len_hbm.at[b], len_v)
    lenb = len_v[...]

    zv = jnp.zeros((L,), jnp.float32)
    for r in range(ZR):
        for k in range(D // L):
            zbuf[r, pl.ds(k * L, L)] = zv

    full = []
    empty = []
    bnd = []
    for c in range(NCH):
        s = pos0 + c * CH
        f = jnp.all(lenb >= s + CH)
        e = jnp.all(lenb <= s)
        full.append(f)
        empty.append(e)
        bnd.append(jnp.logical_not(f) & jnp.logical_not(e))

    lane = lax.iota(jnp.int32, L)
    for c in range(NCH):
        for i in range(CH // L):
            vals = lane + (pos0 + c * CH + i * L + 1)
            idx_v[c, pl.ds(i * L, L)] = jnp.where(vals <= lenb, vals, 0)

    bufs = (buf0, buf1, buf2)
    gsems = (gsem0, gsem1, gsem2)
    psems = (psem0, psem1, psem2)

    def lin_fill(c):
        return pltpu.make_async_copy(
            tabs_hbm.at[pl.ds(pos0 + c * CH, CH)], bufs[c % NBUF],
            gsems[c % NBUF])

    def ind_fill(c):
        return pltpu.make_async_copy(
            table_hbm.at[idx_v.at[c]], bufs[c % NBUF], gsems[c % NBUF])

    def put_data(c):
        return pltpu.make_async_copy(
            bufs[c % NBUF], out_hbm.at[b, pl.ds(pos0 + c * CH, CH)],
            psems[c % NBUF])

    def put_zero(c, h):
        return pltpu.make_async_copy(
            zbuf, out_hbm.at[b, pl.ds(pos0 + c * CH + h * ZR, ZR)], zsem)

    def fill(c):
        @pl.when(full[c])
        def _():
            lin_fill(c).start()

        @pl.when(bnd[c])
        def _():
            ind_fill(c).start()

    def wait_fill(c):
        @pl.when(jnp.logical_not(empty[c]))
        def _():
            lin_fill(c).wait()

    def put(c):
        @pl.when(jnp.logical_not(empty[c]))
        def _():
            put_data(c).start()

        @pl.when(empty[c])
        def _():
            put_zero(c, 0).start()
            put_zero(c, 1).start()

    def wait_put(c):
        @pl.when(jnp.logical_not(empty[c]))
        def _():
            put_data(c).wait()

    fill(0)
    fill(1)
    for c in range(NCH):
        if c + 2 < NCH:
            if c >= 1:
                wait_put(c - 1)
            fill(c + 2)
        wait_fill(c)
        put(c)
    wait_put(NCH - 3)
    wait_put(NCH - 2)
    wait_put(NCH - 1)
    for c in range(NCH):
        @pl.when(empty[c])
        def _():
            put_zero(c, 0).wait()
            put_zero(c, 1).wait()


def kernel(input_len, pos_enc):
    len_bcast = jnp.broadcast_to(input_len.astype(jnp.int32)[:, None], (B, L))
    tab_shift = lax.slice(pos_enc, (1, 0), (MAX_LEN + 1, D))
    mesh = plsc.VectorSubcoreMesh(core_axis_name="c", subcore_axis_name="s")
    run = functools.partial(
        pl.kernel,
        mesh=mesh,
        out_type=jax.ShapeDtypeStruct((B, MAX_LEN, D), jnp.float32),
        compiler_params=pltpu.CompilerParams(needs_layout_passes=False),
        scratch_types=[
            pltpu.VMEM((L,), jnp.int32),
            pltpu.VMEM((NCH, CH), jnp.int32),
            pltpu.VMEM((CH, D), jnp.float32),
            pltpu.VMEM((CH, D), jnp.float32),
            pltpu.VMEM((CH, D), jnp.float32),
            pltpu.VMEM((ZR, D), jnp.float32),
            pltpu.SemaphoreType.DMA,
            pltpu.SemaphoreType.DMA,
            pltpu.SemaphoreType.DMA,
            pltpu.SemaphoreType.DMA,
            pltpu.SemaphoreType.DMA,
            pltpu.SemaphoreType.DMA,
            pltpu.SemaphoreType.DMA,
        ],
    )(_pe_body)
    return run(len_bcast, tab_shift, pos_enc)


# speedup vs baseline: 3.3063x; 1.0124x over previous
import functools

import jax
import jax.numpy as jnp
from jax import lax
from jax.experimental import pallas as pl
from jax.experimental.pallas import tpu as pltpu
from jax.experimental.pallas import tpu_sc as plsc

MAX_SEQ_LEN = 20480
D = 1024
MAX_LEN = 2048
B = 4

_INFO = plsc.get_sparse_core_info()
NC = _INFO.num_cores
NS = _INFO.num_subcores
L = _INFO.num_lanes
NW = NC * NS

ROWS = B * MAX_LEN
RPT = ROWS // NW
TPB = MAX_LEN // RPT
CH = 32
NCH = RPT // CH
NBUF = 3
ZR = 8


def _pe_body(len_hbm, tabs_hbm, table_hbm, out_hbm, len_v, idx_v,
             buf0, buf1, buf2, zbuf, gsem0, gsem1, gsem2, psem0, psem1, psem2,
             zsem):
    wid = lax.axis_index("s") * NC + lax.axis_index("c")
    b = wid // TPB
    pos0 = (wid % TPB) * RPT

    pltpu.sync_copy(len_hbm.at[b], len_v)
    lenb = len_v[...]

    zv = jnp.zeros((L,), jnp.float32)
    for r in range(ZR):
        for k in range(D // L):
            zbuf[r, pl.ds(k * L, L)] = zv

    full = []
    empty = []
    bnd = []
    for c in range(NCH):
        s = pos0 + c * CH
        f = jnp.all(lenb >= s + CH)
        e = jnp.all(lenb <= s)
        full.append(f)
        empty.append(e)
        bnd.append(jnp.logical_not(f) & jnp.logical_not(e))

    lane = lax.iota(jnp.int32, L)
    for c in range(NCH):
        for i in range(CH // L):
            vals = lane + (pos0 + c * CH + i * L + 1)
            idx_v[c, pl.ds(i * L, L)] = jnp.where(vals <= lenb, vals, 0)

    bufs = (buf0, buf1, buf2)
    gsems = (gsem0, gsem1, gsem2)
    psems = (psem0, psem1, psem2)

    def lin_fill(c):
        return pltpu.make_async_copy(
            tabs_hbm.at[pl.ds(pos0 + c * CH, CH)], bufs[c % NBUF],
            gsems[c % NBUF])

    def ind_fill(c):
        return pltpu.make_async_copy(
            table_hbm.at[idx_v.at[c]], bufs[c % NBUF], gsems[c % NBUF])

    def put_data(c):
        return pltpu.make_async_copy(
            bufs[c % NBUF], out_hbm.at[b, pl.ds(pos0 + c * CH, CH)],
            psems[c % NBUF])

    def put_zero(c, h):
        return pltpu.make_async_copy(
            zbuf, out_hbm.at[b, pl.ds(pos0 + c * CH + h * ZR, ZR)], zsem)

    def fill(c):
        @pl.when(full[c])
        def _():
            lin_fill(c).start()

        @pl.when(bnd[c])
        def _():
            ind_fill(c).start()

    def wait_fill(c):
        @pl.when(jnp.logical_not(empty[c]))
        def _():
            lin_fill(c).wait()

    def put(c):
        @pl.when(jnp.logical_not(empty[c]))
        def _():
            put_data(c).start()

        @pl.when(empty[c])
        def _():
            for h in range(CH // ZR):
                put_zero(c, h).start()

    def wait_put(c):
        @pl.when(jnp.logical_not(empty[c]))
        def _():
            put_data(c).wait()

    fill(0)
    fill(1)
    for c in range(NCH):
        if c + 2 < NCH:
            if c >= 1:
                wait_put(c - 1)
            fill(c + 2)
        wait_fill(c)
        put(c)
    wait_put(NCH - 3)
    wait_put(NCH - 2)
    wait_put(NCH - 1)
    for c in range(NCH):
        @pl.when(empty[c])
        def _():
            for h in range(CH // ZR):
                put_zero(c, h).wait()


def kernel(input_len, pos_enc):
    len_bcast = jnp.broadcast_to(input_len.astype(jnp.int32)[:, None], (B, L))
    tab_shift = lax.slice(pos_enc, (1, 0), (MAX_LEN + 1, D))
    mesh = plsc.VectorSubcoreMesh(core_axis_name="c", subcore_axis_name="s")
    run = functools.partial(
        pl.kernel,
        mesh=mesh,
        out_type=jax.ShapeDtypeStruct((B, MAX_LEN, D), jnp.float32),
        compiler_params=pltpu.CompilerParams(needs_layout_passes=False),
        scratch_types=[
            pltpu.VMEM((L,), jnp.int32),
            pltpu.VMEM((NCH, CH), jnp.int32),
            pltpu.VMEM((CH, D), jnp.float32),
            pltpu.VMEM((CH, D), jnp.float32),
            pltpu.VMEM((CH, D), jnp.float32),
            pltpu.VMEM((ZR, D), jnp.float32),
            pltpu.SemaphoreType.DMA,
            pltpu.SemaphoreType.DMA,
            pltpu.SemaphoreType.DMA,
            pltpu.SemaphoreType.DMA,
            pltpu.SemaphoreType.DMA,
            pltpu.SemaphoreType.DMA,
            pltpu.SemaphoreType.DMA,
        ],
    )(_pe_body)
    return run(len_bcast, tab_shift, pos_enc)
